# baseline (speedup 1.0000x reference)
import jax
import jax.numpy as jnp
from jax.experimental import pallas as pl
from jax.experimental.pallas import tpu as pltpu

N_TOK = 8192
K_CODE = 8192
D = 256

WPAD = 4096
NW = 2

TILE_TA = 512
NTA = N_TOK // TILE_TA

TILE_T = 256
TILE_KB = 512
NT = N_TOK // TILE_T
NKB = K_CODE // TILE_KB

_BIG_I32 = 2 ** 30


def _argmin_body(xsq_ref, wsq_ref, x_ref, w_ref, idx_ref, minv_ref, mini_ref):
    k = pl.program_id(1)
    m = jax.lax.dot_general(x_ref[...], w_ref[...], (((1,), (1,)), ((), ())),
                            preferred_element_type=jnp.float32)
    d = (xsq_ref[...] + wsq_ref[...]) - 2.0 * m
    rowmin = jnp.min(d, axis=1, keepdims=True)
    lid = jax.lax.broadcasted_iota(jnp.int32, d.shape, 1)
    rowidx = jnp.min(jnp.where(d == rowmin, lid, _BIG_I32),
                     axis=1, keepdims=True) + k * WPAD

    @pl.when(k == 0)
    def _():
        minv_ref[...] = rowmin.astype(jnp.bfloat16).astype(jnp.float32)
        mini_ref[...] = rowidx

    @pl.when(k != 0)
    def _():
        lt = rowmin < minv_ref[...]
        mini_ref[...] = jnp.where(lt, rowidx, mini_ref[...])
        minv_ref[...] = jnp.where(lt, rowmin, minv_ref[...]).astype(
            jnp.bfloat16).astype(jnp.float32)

    @pl.when(k == NW - 1)
    def _():
        idx_ref[...] = mini_ref[...]


def _assemble_body(idx_ref, x_ref, w_ref, enc_ref, qst_ref, loss_ref, perp_ref,
                   qacc_ref, cnt_ref, sq_ref):
    t = pl.program_id(0)
    k = pl.program_id(1)
    idx = idx_ref[...]
    cols = jax.lax.broadcasted_iota(jnp.int32, (TILE_T, TILE_KB), 1) + k * TILE_KB
    onehot = (idx == cols).astype(jnp.float32)
    enc_ref[...] = onehot

    part = jax.lax.dot_general(onehot.astype(jnp.bfloat16), w_ref[...],
                               (((1,), (0,)), ((), ())),
                               preferred_element_type=jnp.float32)

    @pl.when(k == 0)
    def _():
        qacc_ref[...] = part

    @pl.when(k != 0)
    def _():
        qacc_ref[...] += part

    csum = jnp.sum(onehot, axis=0)

    @pl.when(t == 0)
    def _():
        cnt_ref[pl.ds(k * TILE_KB, TILE_KB)] = csum

    @pl.when(t != 0)
    def _():
        cnt_ref[pl.ds(k * TILE_KB, TILE_KB)] += csum

    @pl.when(k == NKB - 1)
    def _():
        x = x_ref[...]
        q = qacc_ref[...]
        qst_ref[...] = x + (q - x)
        dlt = q - x
        psum = jnp.sum(dlt * dlt)

        @pl.when(t == 0)
        def _():
            sq_ref[0, 0] = psum

        @pl.when(t != 0)
        def _():
            sq_ref[0, 0] += psum

    @pl.when(jnp.logical_and(t == NT - 1, k == NKB - 1))
    def _():
        mse = sq_ref[0, 0] / jnp.float32(N_TOK * D)
        loss_ref[...] = jnp.reshape(1.25 * mse, (1, 1))
        p = cnt_ref[...] * jnp.float32(1.0 / N_TOK)
        ent = jnp.sum(p * jnp.log(p + 1e-10))
        perp_ref[...] = jnp.reshape(jnp.exp(-ent), (1, 1))


def kernel(inputs, weight):
    input_shape = inputs.shape
    flat = inputs.reshape(-1, D)
    xsq = jnp.sum(flat ** 2, axis=1, keepdims=True)
    wsq = jnp.sum(weight ** 2, axis=1)[None, :]

    w_pad = weight.astype(jnp.bfloat16)
    x_bf = flat.astype(jnp.bfloat16)

    idx = pl.pallas_call(
        _argmin_body,
        grid=(NTA, NW),
        in_specs=[
            pl.BlockSpec((TILE_TA, 1), lambda t, k: (t, 0)),
            pl.BlockSpec((1, WPAD), lambda t, k: (0, k)),
            pl.BlockSpec((TILE_TA, D), lambda t, k: (t, 0)),
            pl.BlockSpec((WPAD, D), lambda t, k: (k, 0)),
        ],
        out_specs=pl.BlockSpec((TILE_TA, 1), lambda t, k: (t, 0)),
        out_shape=jax.ShapeDtypeStruct((N_TOK, 1), jnp.int32),
        scratch_shapes=[
            pltpu.VMEM((TILE_TA, 1), jnp.float32),
            pltpu.VMEM((TILE_TA, 1), jnp.int32),
        ],
        compiler_params=pltpu.CompilerParams(
            dimension_semantics=("arbitrary", "arbitrary")),
    )(xsq, wsq, x_bf, w_pad)

    w_bf = weight.astype(jnp.bfloat16)
    enc, qst, loss, perp = pl.pallas_call(
        _assemble_body,
        grid=(NT, NKB),
        in_specs=[
            pl.BlockSpec((TILE_T, 1), lambda t, k: (t, 0)),
            pl.BlockSpec((TILE_T, D), lambda t, k: (t, 0)),
            pl.BlockSpec((TILE_KB, D), lambda t, k: (k, 0)),
        ],
        out_specs=[
            pl.BlockSpec((TILE_T, TILE_KB), lambda t, k: (t, k)),
            pl.BlockSpec((TILE_T, D), lambda t, k: (t, 0)),
            pl.BlockSpec((1, 1), lambda t, k: (0, 0)),
            pl.BlockSpec((1, 1), lambda t, k: (0, 0)),
        ],
        out_shape=[
            jax.ShapeDtypeStruct((N_TOK, K_CODE), jnp.float32),
            jax.ShapeDtypeStruct((N_TOK, D), jnp.float32),
            jax.ShapeDtypeStruct((1, 1), jnp.float32),
            jax.ShapeDtypeStruct((1, 1), jnp.float32),
        ],
        scratch_shapes=[
            pltpu.VMEM((TILE_T, D), jnp.float32),
            pltpu.VMEM((K_CODE,), jnp.float32),
            pltpu.SMEM((1, 1), jnp.float32),
        ],
        compiler_params=pltpu.CompilerParams(
            dimension_semantics=("arbitrary", "arbitrary")),
    )(idx, flat, w_bf)

    loss = loss[0, 0]
    perplexity = perp[0, 0]
    quantized_st = qst.reshape(input_shape)
    return (loss, quantized_st, perplexity, enc)


# speedup vs baseline: 1.9712x; 1.9712x over previous
import jax
import jax.numpy as jnp
from jax.experimental import pallas as pl
from jax.experimental.pallas import tpu as pltpu

N_TOK = 8192
K_CODE = 8192
D = 256

WPAD = 4096
NW = 2

TILE_TA = 512
NTA = N_TOK // TILE_TA

TILE_T = 256
TILE_KB = 2048
NT = N_TOK // TILE_T
NKB = K_CODE // TILE_KB

_BIG_I32 = 2 ** 30


def _argmin_body(xsq_ref, wsq_ref, x_ref, w_ref, idx_ref, minv_ref, mini_ref):
    k = pl.program_id(1)
    m = jax.lax.dot_general(x_ref[...], w_ref[...], (((1,), (1,)), ((), ())),
                            preferred_element_type=jnp.float32)
    d = (xsq_ref[...] + wsq_ref[...]) - 2.0 * m
    rowmin = jnp.min(d, axis=1, keepdims=True)
    lid = jax.lax.broadcasted_iota(jnp.int32, d.shape, 1)
    rowidx = jnp.min(jnp.where(d == rowmin, lid, _BIG_I32),
                     axis=1, keepdims=True) + k * WPAD

    @pl.when(k == 0)
    def _():
        minv_ref[...] = rowmin.astype(jnp.bfloat16).astype(jnp.float32)
        mini_ref[...] = rowidx

    @pl.when(k != 0)
    def _():
        lt = rowmin < minv_ref[...]
        mini_ref[...] = jnp.where(lt, rowidx, mini_ref[...])
        minv_ref[...] = jnp.where(lt, rowmin, minv_ref[...]).astype(
            jnp.bfloat16).astype(jnp.float32)

    @pl.when(k == NW - 1)
    def _():
        idx_ref[...] = mini_ref[...]


def _assemble_body(idx_ref, x_ref, w_ref, enc_ref, qst_ref, loss_ref, perp_ref,
                   qacc_ref, cnt_ref, sq_ref):
    t = pl.program_id(0)
    k = pl.program_id(1)
    idx = idx_ref[...]
    cols = jax.lax.broadcasted_iota(jnp.int32, (TILE_T, TILE_KB), 1) + k * TILE_KB
    onehot = (idx == cols).astype(jnp.float32)
    enc_ref[...] = onehot

    part = jax.lax.dot_general(onehot.astype(jnp.bfloat16),
                               w_ref[pl.ds(k * TILE_KB, TILE_KB), :],
                               (((1,), (0,)), ((), ())),
                               preferred_element_type=jnp.float32)

    @pl.when(k == 0)
    def _():
        qacc_ref[...] = part

    @pl.when(k != 0)
    def _():
        qacc_ref[...] += part

    csum = jnp.sum(onehot, axis=0)

    @pl.when(t == 0)
    def _():
        cnt_ref[pl.ds(k * TILE_KB, TILE_KB)] = csum

    @pl.when(t != 0)
    def _():
        cnt_ref[pl.ds(k * TILE_KB, TILE_KB)] += csum

    @pl.when(k == NKB - 1)
    def _():
        x = x_ref[...]
        q = qacc_ref[...]
        qst_ref[...] = x + (q - x)
        dlt = q - x
        psum = jnp.sum(dlt * dlt)

        @pl.when(t == 0)
        def _():
            sq_ref[0, 0] = psum

        @pl.when(t != 0)
        def _():
            sq_ref[0, 0] += psum

    @pl.when(jnp.logical_and(t == NT - 1, k == NKB - 1))
    def _():
        mse = sq_ref[0, 0] / jnp.float32(N_TOK * D)
        loss_ref[...] = jnp.reshape(1.25 * mse, (1, 1))
        p = cnt_ref[...] * jnp.float32(1.0 / N_TOK)
        ent = jnp.sum(p * jnp.log(p + 1e-10))
        perp_ref[...] = jnp.reshape(jnp.exp(-ent), (1, 1))


def kernel(inputs, weight):
    input_shape = inputs.shape
    flat = inputs.reshape(-1, D)
    xsq = jnp.sum(flat ** 2, axis=1, keepdims=True)
    wsq = jnp.sum(weight ** 2, axis=1)[None, :]

    w_pad = weight.astype(jnp.bfloat16)
    x_bf = flat.astype(jnp.bfloat16)

    idx = pl.pallas_call(
        _argmin_body,
        grid=(NTA, NW),
        in_specs=[
            pl.BlockSpec((TILE_TA, 1), lambda t, k: (t, 0)),
            pl.BlockSpec((1, WPAD), lambda t, k: (0, k)),
            pl.BlockSpec((TILE_TA, D), lambda t, k: (t, 0)),
            pl.BlockSpec((WPAD, D), lambda t, k: (k, 0)),
        ],
        out_specs=pl.BlockSpec((TILE_TA, 1), lambda t, k: (t, 0)),
        out_shape=jax.ShapeDtypeStruct((N_TOK, 1), jnp.int32),
        scratch_shapes=[
            pltpu.VMEM((TILE_TA, 1), jnp.float32),
            pltpu.VMEM((TILE_TA, 1), jnp.int32),
        ],
        compiler_params=pltpu.CompilerParams(
            dimension_semantics=("arbitrary", "arbitrary")),
    )(xsq, wsq, x_bf, w_pad)

    w_bf = weight.astype(jnp.bfloat16)
    enc, qst, loss, perp = pl.pallas_call(
        _assemble_body,
        grid=(NT, NKB),
        in_specs=[
            pl.BlockSpec((TILE_T, 1), lambda t, k: (t, 0)),
            pl.BlockSpec((TILE_T, D), lambda t, k: (t, 0)),
            pl.BlockSpec((K_CODE, D), lambda t, k: (0, 0)),
        ],
        out_specs=[
            pl.BlockSpec((TILE_T, TILE_KB), lambda t, k: (t, k)),
            pl.BlockSpec((TILE_T, D), lambda t, k: (t, 0)),
            pl.BlockSpec((1, 1), lambda t, k: (0, 0)),
            pl.BlockSpec((1, 1), lambda t, k: (0, 0)),
        ],
        out_shape=[
            jax.ShapeDtypeStruct((N_TOK, K_CODE), jnp.float32),
            jax.ShapeDtypeStruct((N_TOK, D), jnp.float32),
            jax.ShapeDtypeStruct((1, 1), jnp.float32),
            jax.ShapeDtypeStruct((1, 1), jnp.float32),
        ],
        scratch_shapes=[
            pltpu.VMEM((TILE_T, D), jnp.float32),
            pltpu.VMEM((K_CODE,), jnp.float32),
            pltpu.SMEM((1, 1), jnp.float32),
        ],
        compiler_params=pltpu.CompilerParams(
            dimension_semantics=("arbitrary", "arbitrary")),
    )(idx, flat, w_bf)

    loss = loss[0, 0]
    perplexity = perp[0, 0]
    quantized_st = qst.reshape(input_shape)
    return (loss, quantized_st, perplexity, enc)


# speedup vs baseline: 2.2286x; 1.1306x over previous
import jax
import jax.numpy as jnp
from jax.experimental import pallas as pl
from jax.experimental.pallas import tpu as pltpu

N_TOK = 8192
K_CODE = 8192
D = 256

WPAD = 4096
NW = 2

TILE_TA = 512
NTA = N_TOK // TILE_TA

TILE_T = 512
TILE_KB = 2048
NT = N_TOK // TILE_T
NKB = K_CODE // TILE_KB

_BIG_I32 = 2 ** 30


def _argmin_body(xsq_ref, wsq_ref, x_ref, w_ref, idx_ref, minv_ref, mini_ref):
    k = pl.program_id(1)
    m = jax.lax.dot_general(x_ref[...], w_ref[...], (((1,), (1,)), ((), ())),
                            preferred_element_type=jnp.float32)
    d = (xsq_ref[...] + wsq_ref[...]) - 2.0 * m
    rowmin = jnp.min(d, axis=1, keepdims=True)
    lid = jax.lax.broadcasted_iota(jnp.int32, d.shape, 1)
    rowidx = jnp.min(jnp.where(d == rowmin, lid, _BIG_I32),
                     axis=1, keepdims=True) + k * WPAD

    @pl.when(k == 0)
    def _():
        minv_ref[...] = rowmin.astype(jnp.bfloat16).astype(jnp.float32)
        mini_ref[...] = rowidx

    @pl.when(k != 0)
    def _():
        lt = rowmin < minv_ref[...]
        mini_ref[...] = jnp.where(lt, rowidx, mini_ref[...])
        minv_ref[...] = jnp.where(lt, rowmin, minv_ref[...]).astype(
            jnp.bfloat16).astype(jnp.float32)

    @pl.when(k == NW - 1)
    def _():
        idx_ref[...] = mini_ref[...]


def _assemble_body(idx_ref, x_ref, w_ref, enc_ref, qst_ref, loss_ref, perp_ref,
                   qacc_ref, cnt_ref, sq_ref):
    t = pl.program_id(0)
    k = pl.program_id(1)
    idx = idx_ref[...]
    cols = jax.lax.broadcasted_iota(jnp.int32, (TILE_T, TILE_KB), 1) + k * TILE_KB
    onehot = (idx == cols).astype(jnp.float32)
    enc_ref[...] = onehot

    part = jax.lax.dot_general(onehot.astype(jnp.bfloat16),
                               w_ref[pl.ds(k * TILE_KB, TILE_KB), :],
                               (((1,), (0,)), ((), ())),
                               preferred_element_type=jnp.float32)

    @pl.when(k == 0)
    def _():
        qacc_ref[...] = part

    @pl.when(k != 0)
    def _():
        qacc_ref[...] += part

    csum = jnp.sum(onehot, axis=0)

    @pl.when(t == 0)
    def _():
        cnt_ref[pl.ds(k * TILE_KB, TILE_KB)] = csum

    @pl.when(t != 0)
    def _():
        cnt_ref[pl.ds(k * TILE_KB, TILE_KB)] += csum

    @pl.when(k == NKB - 1)
    def _():
        x = x_ref[...]
        q = qacc_ref[...]
        qst_ref[...] = x + (q - x)
        dlt = q - x
        psum = jnp.sum(dlt * dlt)

        @pl.when(t == 0)
        def _():
            sq_ref[0, 0] = psum

        @pl.when(t != 0)
        def _():
            sq_ref[0, 0] += psum

    @pl.when(jnp.logical_and(t == NT - 1, k == NKB - 1))
    def _():
        mse = sq_ref[0, 0] / jnp.float32(N_TOK * D)
        loss_ref[...] = jnp.reshape(1.25 * mse, (1, 1))
        p = cnt_ref[...] * jnp.float32(1.0 / N_TOK)
        ent = jnp.sum(p * jnp.log(p + 1e-10))
        perp_ref[...] = jnp.reshape(jnp.exp(-ent), (1, 1))


def kernel(inputs, weight):
    input_shape = inputs.shape
    flat = inputs.reshape(-1, D)
    xsq = jnp.sum(flat ** 2, axis=1, keepdims=True)
    wsq = jnp.sum(weight ** 2, axis=1)[None, :]

    w_pad = weight.astype(jnp.bfloat16)
    x_bf = flat.astype(jnp.bfloat16)

    idx = pl.pallas_call(
        _argmin_body,
        grid=(NTA, NW),
        in_specs=[
            pl.BlockSpec((TILE_TA, 1), lambda t, k: (t, 0)),
            pl.BlockSpec((1, WPAD), lambda t, k: (0, k)),
            pl.BlockSpec((TILE_TA, D), lambda t, k: (t, 0)),
            pl.BlockSpec((WPAD, D), lambda t, k: (k, 0)),
        ],
        out_specs=pl.BlockSpec((TILE_TA, 1), lambda t, k: (t, 0)),
        out_shape=jax.ShapeDtypeStruct((N_TOK, 1), jnp.int32),
        scratch_shapes=[
            pltpu.VMEM((TILE_TA, 1), jnp.float32),
            pltpu.VMEM((TILE_TA, 1), jnp.int32),
        ],
        compiler_params=pltpu.CompilerParams(
            dimension_semantics=("arbitrary", "arbitrary")),
    )(xsq, wsq, x_bf, w_pad)

    w_bf = weight.astype(jnp.bfloat16)
    enc, qst, loss, perp = pl.pallas_call(
        _assemble_body,
        grid=(NT, NKB),
        in_specs=[
            pl.BlockSpec((TILE_T, 1), lambda t, k: (t, 0)),
            pl.BlockSpec((TILE_T, D), lambda t, k: (t, 0)),
            pl.BlockSpec((K_CODE, D), lambda t, k: (0, 0)),
        ],
        out_specs=[
            pl.BlockSpec((TILE_T, TILE_KB), lambda t, k: (t, k)),
            pl.BlockSpec((TILE_T, D), lambda t, k: (t, 0)),
            pl.BlockSpec((1, 1), lambda t, k: (0, 0)),
            pl.BlockSpec((1, 1), lambda t, k: (0, 0)),
        ],
        out_shape=[
            jax.ShapeDtypeStruct((N_TOK, K_CODE), jnp.float32),
            jax.ShapeDtypeStruct((N_TOK, D), jnp.float32),
            jax.ShapeDtypeStruct((1, 1), jnp.float32),
            jax.ShapeDtypeStruct((1, 1), jnp.float32),
        ],
        scratch_shapes=[
            pltpu.VMEM((TILE_T, D), jnp.float32),
            pltpu.VMEM((K_CODE,), jnp.float32),
            pltpu.SMEM((1, 1), jnp.float32),
        ],
        compiler_params=pltpu.CompilerParams(
            dimension_semantics=("arbitrary", "arbitrary")),
    )(idx, flat, w_bf)

    loss = loss[0, 0]
    perplexity = perp[0, 0]
    quantized_st = qst.reshape(input_shape)
    return (loss, quantized_st, perplexity, enc)
